# Initial kernel scaffold; baseline (speedup 1.0000x reference)
import jax
import jax.numpy as jnp
from jax.experimental import pallas as pl


def kernel(x, edge_index, W_src0, W_dst0, attn0, bias0, res0, W_src1, W_dst1, attn1, bias1, gate_W, gate_b, lin_W, lin_b, cls_W, cls_b):
    raise NotImplementedError("write your pallas kernel here")


# speedup vs baseline: 43.5633x; 43.5633x over previous
import functools

import jax
import jax.numpy as jnp
from jax import lax
from jax.experimental import pallas as pl
from jax.experimental.pallas import tpu as pltpu
from jax.experimental.pallas import tpu_sc as plsc

_NW = 32
_C = 128
_RB = 112


def _edge_sc(fs0, fs1, fd0, fd1, src, dst, attn):
  n = fs0.shape[0]
  e = src.shape[0]
  epw = e // _NW
  nchunks = epw // _C
  tail = epw - nchunks * _C
  rpt = -(-(-(-n // 16)) // _RB) * _RB
  npad = 16 * rpt
  nrb = rpt // _RB

  mesh = plsc.VectorSubcoreMesh(core_axis_name="c", subcore_axis_name="s")

  def body(fs0_h, fs1_h, fd0_h, fd1_h, src_h, dst_h, attn_h, out_h,
           table, srcv, dstv, fsr, fdr, contrib,
           srcvt, dstvt, fsrt, fdrt, contribt,
           zbuf, tmpb, attnv, sem1, sem2):
    cid = lax.axis_index("c")
    sid = lax.axis_index("s")
    wid = sid * 2 + cid
    ebase = wid * epw
    tbase = sid * rpt

    ii = lax.iota(jnp.int32, 16)
    onehot = jnp.where(ii == 0, 1.0, 0.0).astype(jnp.float32)
    xors = [ii ^ sh for sh in (1, 2, 4, 8)]

    def _dyng(t, idx):
      dn = lax.GatherDimensionNumbers(
          offset_dims=(), collapsed_slice_dims=(0,), start_index_map=(0,))
      return lax.gather(t, idx[:, None], dn, (1,),
                        mode=lax.GatherScatterMode.PROMISE_IN_BOUNDS)

    def zb(r, c):
      zbuf[r, 0:16] = jnp.zeros((16,), jnp.float32)
      zbuf[r, 16:32] = jnp.zeros((16,), jnp.float32)
      return c
    lax.fori_loop(0, _RB, zb, 0)

    def zero_table():
      for b in range(nrb):
        pltpu.sync_copy(zbuf, table.at[pl.ds(tbase + b * _RB, _RB)])

    def run_phase(h, fs_t, fd_t):
      pltpu.sync_copy(attn_h.at[pl.ds(h * 16, 16)], attnv)
      attn_vec = attnv[...]

      def do_chunk(base, sz, sv, dv, fr, dr, cb):
        pltpu.sync_copy(src_h.at[pl.ds(base, sz)], sv)
        pltpu.sync_copy(dst_h.at[pl.ds(base, sz)], dv)
        cp1 = pltpu.async_copy(fs_t.at[sv], fr, sem1)
        cp2 = pltpu.async_copy(fd_t.at[dv], dr, sem2)
        cp1.wait()
        cp2.wait()

        def ebody(ei, c):
          a = fr[ei, 0:16]
          bb = dr[ei, 0:16]
          s = a + bb
          t = jnp.maximum(s, 0.2 * s) * attn_vec
          for ix in xors:
            t = t + _dyng(t, ix)
          exv = jnp.exp(t)
          cb[ei, 0:16] = exv * a
          cb[ei, 16:32] = exv * onehot
          return c
        lax.fori_loop(0, sz, ebody, 0)
        pltpu.sync_copy(cb, table.at[dv], add=True)

      def chunk_loop(i, c):
        do_chunk(ebase + i * _C, _C, srcv, dstv, fsr, fdr, contrib)
        return c
      lax.fori_loop(0, nchunks, chunk_loop, 0)
      if tail:
        do_chunk(ebase + nchunks * _C, tail, srcvt, dstvt, fsrt, fdrt,
                 contribt)

    def flush(h):
      for b in range(nrb):
        r0 = tbase + b * _RB
        pltpu.sync_copy(table.at[pl.ds(r0, _RB)], tmpb)
        off = (cid * 2 + h) * npad + r0
        pltpu.sync_copy(tmpb, out_h.at[pl.ds(off, _RB)])

    zero_table()
    plsc.subcore_barrier()
    run_phase(0, fs0_h, fd0_h)
    plsc.subcore_barrier()
    flush(0)
    zero_table()
    plsc.subcore_barrier()
    run_phase(1, fs1_h, fd1_h)
    plsc.subcore_barrier()
    flush(1)

  f = pl.kernel(
      body,
      out_type=jax.ShapeDtypeStruct((4 * npad, 32), jnp.float32),
      mesh=mesh,
      compiler_params=pltpu.CompilerParams(use_tc_tiling_on_sc=False),
      scratch_types=[
          pltpu.VMEM_SHARED((npad, 32), jnp.float32),
          pltpu.VMEM((_C,), jnp.int32),
          pltpu.VMEM((_C,), jnp.int32),
          pltpu.VMEM((_C, 16), jnp.float32),
          pltpu.VMEM((_C, 16), jnp.float32),
          pltpu.VMEM((_C, 32), jnp.float32),
          pltpu.VMEM((tail or 8,), jnp.int32),
          pltpu.VMEM((tail or 8,), jnp.int32),
          pltpu.VMEM((tail or 8, 16), jnp.float32),
          pltpu.VMEM((tail or 8, 16), jnp.float32),
          pltpu.VMEM((tail or 8, 32), jnp.float32),
          pltpu.VMEM((_RB, 32), jnp.float32),
          pltpu.VMEM((_RB, 32), jnp.float32),
          pltpu.VMEM((16,), jnp.float32),
          pltpu.SemaphoreType.DMA,
          pltpu.SemaphoreType.DMA,
      ],
  )
  out = f(fs0, fs1, fd0, fd1, src, dst, attn.reshape(-1))
  return out.reshape(2, 2, npad, 32)[:, :, :n, :]


def _mm(x, w):
  n, k = x.shape
  m = w.shape[1]
  blk = 1000
  grid = n // blk
  return pl.pallas_call(
      lambda x_ref, w_ref, o_ref: o_ref.__setitem__(
          (...,), jnp.dot(x_ref[...], w_ref[...],
                          preferred_element_type=jnp.float32)),
      grid=(grid,),
      in_specs=[
          pl.BlockSpec((blk, k), lambda i: (i, 0)),
          pl.BlockSpec((k, m), lambda i: (0, 0)),
      ],
      out_specs=pl.BlockSpec((blk, m), lambda i: (i, 0)),
      out_shape=jax.ShapeDtypeStruct((n, m), jnp.float32),
  )(x, w)


def _combine(acc, rv, bias):
  n = rv.shape[0]
  blk = 1000
  grid = n // blk

  def kfn(acc_ref, rv_ref, b_ref, o_ref, cm_ref):
    i = pl.program_id(0)
    a0 = acc_ref[0, 0] + acc_ref[1, 0]
    a1 = acc_ref[0, 1] + acc_ref[1, 1]
    num = jnp.concatenate([a0[:, 0:16], a1[:, 0:16]], axis=1)
    den = jnp.concatenate([
        jnp.broadcast_to(a0[:, 16:17], (blk, 16)),
        jnp.broadcast_to(a1[:, 16:17], (blk, 16)),
    ], axis=1)
    h = jnp.maximum(num / (den + 1e-16) + rv_ref[...] + b_ref[...], 0.0)
    o_ref[...] = h
    bm = jnp.max(h, axis=0, keepdims=True)

    @pl.when(i == 0)
    def _():
      cm_ref[...] = bm

    @pl.when(i != 0)
    def _():
      cm_ref[...] = jnp.maximum(cm_ref[...], bm)

  return pl.pallas_call(
      kfn,
      grid=(grid,),
      in_specs=[
          pl.BlockSpec((2, 2, blk, 32), lambda i: (0, 0, i, 0)),
          pl.BlockSpec((blk, 32), lambda i: (i, 0)),
          pl.BlockSpec((1, 32), lambda i: (0, 0)),
      ],
      out_specs=[
          pl.BlockSpec((blk, 32), lambda i: (i, 0)),
          pl.BlockSpec((1, 32), lambda i: (0, 0)),
      ],
      out_shape=[
          jax.ShapeDtypeStruct((n, 32), jnp.float32),
          jax.ShapeDtypeStruct((1, 32), jnp.float32),
      ],
  )(acc, rv, bias)


def _colsumexp(h, cm):
  n = h.shape[0]
  blk = 1000
  grid = n // blk

  def kfn(h_ref, cm_ref, cs_ref):
    i = pl.program_id(0)
    s = jnp.sum(jnp.exp(h_ref[...] - cm_ref[...]), axis=0, keepdims=True)

    @pl.when(i == 0)
    def _():
      cs_ref[...] = s

    @pl.when(i != 0)
    def _():
      cs_ref[...] = cs_ref[...] + s

  return pl.pallas_call(
      kfn,
      grid=(grid,),
      in_specs=[
          pl.BlockSpec((blk, 32), lambda i: (i, 0)),
          pl.BlockSpec((1, 32), lambda i: (0, 0)),
      ],
      out_specs=pl.BlockSpec((1, 32), lambda i: (0, 0)),
      out_shape=jax.ShapeDtypeStruct((1, 32), jnp.float32),
  )(h, cm)


def _gate(h, cm, cs, gw):
  n = h.shape[0]
  blk = 1000
  grid = n // blk

  def kfn(h_ref, cm_ref, cs_ref, gw_ref, g_ref, gm_ref):
    i = pl.program_id(0)
    w = gw_ref[...] / cs_ref[...]
    g = jnp.sum(jnp.exp(h_ref[...] - cm_ref[...]) * w, axis=1, keepdims=True)
    g_ref[...] = g
    bm = jnp.max(g, axis=0, keepdims=True)[:, 0:1]

    @pl.when(i == 0)
    def _():
      gm_ref[...] = bm

    @pl.when(i != 0)
    def _():
      gm_ref[...] = jnp.maximum(gm_ref[...], bm)

  return pl.pallas_call(
      kfn,
      grid=(grid,),
      in_specs=[
          pl.BlockSpec((blk, 32), lambda i: (i, 0)),
          pl.BlockSpec((1, 32), lambda i: (0, 0)),
          pl.BlockSpec((1, 32), lambda i: (0, 0)),
          pl.BlockSpec((1, 32), lambda i: (0, 0)),
      ],
      out_specs=[
          pl.BlockSpec((blk, 1), lambda i: (i, 0)),
          pl.BlockSpec((1, 1), lambda i: (0, 0)),
      ],
      out_shape=[
          jax.ShapeDtypeStruct((n, 1), jnp.float32),
          jax.ShapeDtypeStruct((1, 1), jnp.float32),
      ],
  )(h, cm, cs, gw)


def _poolsum(h, cm, g, gm):
  n = h.shape[0]
  blk = 1000
  grid = n // blk

  def kfn(h_ref, cm_ref, g_ref, gm_ref, num_ref, den_ref):
    i = pl.program_id(0)
    eg = jnp.exp(g_ref[...] - gm_ref[...])
    nu = jnp.sum(eg * jnp.exp(h_ref[...] - cm_ref[...]), axis=0, keepdims=True)
    de = jnp.sum(eg, axis=0, keepdims=True)

    @pl.when(i == 0)
    def _():
      num_ref[...] = nu
      den_ref[...] = de

    @pl.when(i != 0)
    def _():
      num_ref[...] = num_ref[...] + nu
      den_ref[...] = den_ref[...] + de

  return pl.pallas_call(
      kfn,
      grid=(grid,),
      in_specs=[
          pl.BlockSpec((blk, 32), lambda i: (i, 0)),
          pl.BlockSpec((1, 32), lambda i: (0, 0)),
          pl.BlockSpec((blk, 1), lambda i: (i, 0)),
          pl.BlockSpec((1, 1), lambda i: (0, 0)),
      ],
      out_specs=[
          pl.BlockSpec((1, 32), lambda i: (0, 0)),
          pl.BlockSpec((1, 1), lambda i: (0, 0)),
      ],
      out_shape=[
          jax.ShapeDtypeStruct((1, 32), jnp.float32),
          jax.ShapeDtypeStruct((1, 1), jnp.float32),
      ],
  )(h, cm, g, gm)


def _final(num1, den1, cs1, num2, den2, cs2, lin_w, lin_b, cls_w, cls_b):
  def kfn(n1, d1, c1, n2, d2, c2, lw, lb, cw, cb, o_ref):
    hg = n1[...] / (c1[...] * d1[...]) + n2[...] / (c2[...] * d2[...])
    hid = jnp.maximum(
        jnp.dot(hg, lw[...], preferred_element_type=jnp.float32) + lb[...],
        0.0)
    o_ref[...] = jnp.dot(hid, cw[...],
                         preferred_element_type=jnp.float32) + cb[...]

  return pl.pallas_call(
      kfn,
      out_shape=jax.ShapeDtypeStruct((1, cls_w.shape[1]), jnp.float32),
  )(num1, den1, cs1, num2, den2, cs2, lin_w, lin_b, cls_w, cls_b)


@jax.jit
def kernel(x, edge_index, W_src0, W_dst0, attn0, bias0, res0, W_src1, W_dst1,
           attn1, bias1, gate_W, gate_b, lin_W, lin_b, cls_W, cls_b):
  src = edge_index[0]
  dst = edge_index[1]
  gw = gate_W.reshape(1, -1)
  b0 = bias0.reshape(1, -1)
  b1 = bias1.reshape(1, -1)
  lb = lin_b.reshape(1, -1)
  cb = cls_b.reshape(1, -1)

  p0 = _mm(x, jnp.concatenate([W_src0, W_dst0, res0], axis=1))
  acc0 = _edge_sc(p0[:, 0:16], p0[:, 16:32], p0[:, 32:48], p0[:, 48:64],
                  src, dst, attn0)
  h1, cm1 = _combine(acc0, p0[:, 64:96], b0)
  cs1 = _colsumexp(h1, cm1)
  g1, gm1 = _gate(h1, cm1, cs1, gw)
  num1, den1 = _poolsum(h1, cm1, g1, gm1)

  p1 = _mm(h1, jnp.concatenate([W_src1, W_dst1], axis=1))
  acc1 = _edge_sc(p1[:, 0:16], p1[:, 16:32], p1[:, 32:48], p1[:, 48:64],
                  src, dst, attn1)
  h2, cm2 = _combine(acc1, h1, b1)
  cs2 = _colsumexp(h2, cm2)
  g2, gm2 = _gate(h2, cm2, cs2, gw)
  num2, den2 = _poolsum(h2, cm2, g2, gm2)

  return _final(num1, den1, cs1, num2, den2, cs2,
                lin_W, lb, cls_W, cb)


# speedup vs baseline: 84.9763x; 1.9506x over previous
import functools

import jax
import jax.numpy as jnp
from jax import lax
from jax.experimental import pallas as pl
from jax.experimental.pallas import tpu as pltpu
from jax.experimental.pallas import tpu_sc as plsc

_NW = 32
_C = 128
_RB = 112


def _edge_sc(fs0, fs1, fd0, fd1, src, dst, attn):
  n = fs0.shape[0]
  e = src.shape[0]
  epw = e // _NW
  nchunks = epw // _C
  tail = epw - nchunks * _C
  rpt = -(-(-(-n // 16)) // _RB) * _RB
  npad = 16 * rpt
  nrb = rpt // _RB

  mesh = plsc.VectorSubcoreMesh(core_axis_name="c", subcore_axis_name="s")

  def body(fs0_h, fs1_h, fd0_h, fd1_h, src_h, dst_h, attn_h, out_h,
           table, srcv, dstv, fsr, fdr, contrib,
           srcvt, dstvt, fsrt, fdrt, contribt,
           zbuf, tmpb, attnv, sem1, sem2):
    cid = lax.axis_index("c")
    sid = lax.axis_index("s")
    wid = sid * 2 + cid
    ebase = wid * epw
    tbase = sid * rpt

    ii = lax.iota(jnp.int32, 16)
    onehot = jnp.where(ii == 0, 1.0, 0.0).astype(jnp.float32)
    xors = [ii ^ sh for sh in (1, 2, 4, 8)]

    def _dyng(t, idx):
      dn = lax.GatherDimensionNumbers(
          offset_dims=(), collapsed_slice_dims=(0,), start_index_map=(0,))
      return lax.gather(t, idx[:, None], dn, (1,),
                        mode=lax.GatherScatterMode.PROMISE_IN_BOUNDS)

    def zb(r, c):
      zbuf[r, 0:16] = jnp.zeros((16,), jnp.float32)
      zbuf[r, 16:32] = jnp.zeros((16,), jnp.float32)
      return c
    lax.fori_loop(0, _RB, zb, 0)

    def zero_table():
      for b in range(nrb):
        pltpu.sync_copy(zbuf, table.at[pl.ds(tbase + b * _RB, _RB)])

    def run_phase(h, fs_t, fd_t):
      pltpu.sync_copy(attn_h.at[pl.ds(h * 16, 16)], attnv)
      attn_vec = attnv[...]

      def do_chunk(base, sz, sv, dv, fr, dr, cb):
        pltpu.sync_copy(src_h.at[pl.ds(base, sz)], sv)
        pltpu.sync_copy(dst_h.at[pl.ds(base, sz)], dv)
        cp1 = pltpu.async_copy(fs_t.at[sv], fr, sem1)
        cp2 = pltpu.async_copy(fd_t.at[dv], dr, sem2)
        cp1.wait()
        cp2.wait()

        @plsc.parallel_loop(0, sz, unroll=8)
        def _(ei):
          a = fr[ei, 0:16]
          bb = dr[ei, 0:16]
          s = a + bb
          t = jnp.maximum(s, 0.2 * s) * attn_vec
          for ix in xors:
            t = t + _dyng(t, ix)
          exv = jnp.exp(t)
          cb[ei, 0:16] = exv * a
          cb[ei, 16:32] = exv * onehot
        pltpu.sync_copy(cb, table.at[dv], add=True)

      def chunk_loop(i, c):
        do_chunk(ebase + i * _C, _C, srcv, dstv, fsr, fdr, contrib)
        return c
      lax.fori_loop(0, nchunks, chunk_loop, 0)
      if tail:
        do_chunk(ebase + nchunks * _C, tail, srcvt, dstvt, fsrt, fdrt,
                 contribt)

    def flush(h):
      for b in range(nrb):
        r0 = tbase + b * _RB
        pltpu.sync_copy(table.at[pl.ds(r0, _RB)], tmpb)
        off = (cid * 2 + h) * npad + r0
        pltpu.sync_copy(tmpb, out_h.at[pl.ds(off, _RB)])

    zero_table()
    plsc.subcore_barrier()
    run_phase(0, fs0_h, fd0_h)
    plsc.subcore_barrier()
    flush(0)
    zero_table()
    plsc.subcore_barrier()
    run_phase(1, fs1_h, fd1_h)
    plsc.subcore_barrier()
    flush(1)

  f = pl.kernel(
      body,
      out_type=jax.ShapeDtypeStruct((4 * npad, 32), jnp.float32),
      mesh=mesh,
      compiler_params=pltpu.CompilerParams(use_tc_tiling_on_sc=False),
      scratch_types=[
          pltpu.VMEM_SHARED((npad, 32), jnp.float32),
          pltpu.VMEM((_C,), jnp.int32),
          pltpu.VMEM((_C,), jnp.int32),
          pltpu.VMEM((_C, 16), jnp.float32),
          pltpu.VMEM((_C, 16), jnp.float32),
          pltpu.VMEM((_C, 32), jnp.float32),
          pltpu.VMEM((tail or 8,), jnp.int32),
          pltpu.VMEM((tail or 8,), jnp.int32),
          pltpu.VMEM((tail or 8, 16), jnp.float32),
          pltpu.VMEM((tail or 8, 16), jnp.float32),
          pltpu.VMEM((tail or 8, 32), jnp.float32),
          pltpu.VMEM((_RB, 32), jnp.float32),
          pltpu.VMEM((_RB, 32), jnp.float32),
          pltpu.VMEM((16,), jnp.float32),
          pltpu.SemaphoreType.DMA,
          pltpu.SemaphoreType.DMA,
      ],
  )
  out = f(fs0, fs1, fd0, fd1, src, dst, attn.reshape(-1))
  return out.reshape(2, 2, npad, 32)[:, :, :n, :]


def _mm(x, w):
  n, k = x.shape
  m = w.shape[1]
  blk = 1000
  grid = n // blk
  return pl.pallas_call(
      lambda x_ref, w_ref, o_ref: o_ref.__setitem__(
          (...,), jnp.dot(x_ref[...], w_ref[...],
                          preferred_element_type=jnp.float32)),
      grid=(grid,),
      in_specs=[
          pl.BlockSpec((blk, k), lambda i: (i, 0)),
          pl.BlockSpec((k, m), lambda i: (0, 0)),
      ],
      out_specs=pl.BlockSpec((blk, m), lambda i: (i, 0)),
      out_shape=jax.ShapeDtypeStruct((n, m), jnp.float32),
  )(x, w)


def _combine(acc, rv, bias):
  n = rv.shape[0]
  blk = 1000
  grid = n // blk

  def kfn(acc_ref, rv_ref, b_ref, o_ref, cm_ref):
    i = pl.program_id(0)
    a0 = acc_ref[0, 0] + acc_ref[1, 0]
    a1 = acc_ref[0, 1] + acc_ref[1, 1]
    num = jnp.concatenate([a0[:, 0:16], a1[:, 0:16]], axis=1)
    den = jnp.concatenate([
        jnp.broadcast_to(a0[:, 16:17], (blk, 16)),
        jnp.broadcast_to(a1[:, 16:17], (blk, 16)),
    ], axis=1)
    h = jnp.maximum(num / (den + 1e-16) + rv_ref[...] + b_ref[...], 0.0)
    o_ref[...] = h
    bm = jnp.max(h, axis=0, keepdims=True)

    @pl.when(i == 0)
    def _():
      cm_ref[...] = bm

    @pl.when(i != 0)
    def _():
      cm_ref[...] = jnp.maximum(cm_ref[...], bm)

  return pl.pallas_call(
      kfn,
      grid=(grid,),
      in_specs=[
          pl.BlockSpec((2, 2, blk, 32), lambda i: (0, 0, i, 0)),
          pl.BlockSpec((blk, 32), lambda i: (i, 0)),
          pl.BlockSpec((1, 32), lambda i: (0, 0)),
      ],
      out_specs=[
          pl.BlockSpec((blk, 32), lambda i: (i, 0)),
          pl.BlockSpec((1, 32), lambda i: (0, 0)),
      ],
      out_shape=[
          jax.ShapeDtypeStruct((n, 32), jnp.float32),
          jax.ShapeDtypeStruct((1, 32), jnp.float32),
      ],
  )(acc, rv, bias)


def _colsumexp(h, cm):
  n = h.shape[0]
  blk = 1000
  grid = n // blk

  def kfn(h_ref, cm_ref, cs_ref):
    i = pl.program_id(0)
    s = jnp.sum(jnp.exp(h_ref[...] - cm_ref[...]), axis=0, keepdims=True)

    @pl.when(i == 0)
    def _():
      cs_ref[...] = s

    @pl.when(i != 0)
    def _():
      cs_ref[...] = cs_ref[...] + s

  return pl.pallas_call(
      kfn,
      grid=(grid,),
      in_specs=[
          pl.BlockSpec((blk, 32), lambda i: (i, 0)),
          pl.BlockSpec((1, 32), lambda i: (0, 0)),
      ],
      out_specs=pl.BlockSpec((1, 32), lambda i: (0, 0)),
      out_shape=jax.ShapeDtypeStruct((1, 32), jnp.float32),
  )(h, cm)


def _gate(h, cm, cs, gw):
  n = h.shape[0]
  blk = 1000
  grid = n // blk

  def kfn(h_ref, cm_ref, cs_ref, gw_ref, g_ref, gm_ref):
    i = pl.program_id(0)
    w = gw_ref[...] / cs_ref[...]
    g = jnp.sum(jnp.exp(h_ref[...] - cm_ref[...]) * w, axis=1, keepdims=True)
    g_ref[...] = g
    bm = jnp.max(g, axis=0, keepdims=True)[:, 0:1]

    @pl.when(i == 0)
    def _():
      gm_ref[...] = bm

    @pl.when(i != 0)
    def _():
      gm_ref[...] = jnp.maximum(gm_ref[...], bm)

  return pl.pallas_call(
      kfn,
      grid=(grid,),
      in_specs=[
          pl.BlockSpec((blk, 32), lambda i: (i, 0)),
          pl.BlockSpec((1, 32), lambda i: (0, 0)),
          pl.BlockSpec((1, 32), lambda i: (0, 0)),
          pl.BlockSpec((1, 32), lambda i: (0, 0)),
      ],
      out_specs=[
          pl.BlockSpec((blk, 1), lambda i: (i, 0)),
          pl.BlockSpec((1, 1), lambda i: (0, 0)),
      ],
      out_shape=[
          jax.ShapeDtypeStruct((n, 1), jnp.float32),
          jax.ShapeDtypeStruct((1, 1), jnp.float32),
      ],
  )(h, cm, cs, gw)


def _poolsum(h, cm, g, gm):
  n = h.shape[0]
  blk = 1000
  grid = n // blk

  def kfn(h_ref, cm_ref, g_ref, gm_ref, num_ref, den_ref):
    i = pl.program_id(0)
    eg = jnp.exp(g_ref[...] - gm_ref[...])
    nu = jnp.sum(eg * jnp.exp(h_ref[...] - cm_ref[...]), axis=0, keepdims=True)
    de = jnp.sum(eg, axis=0, keepdims=True)

    @pl.when(i == 0)
    def _():
      num_ref[...] = nu
      den_ref[...] = de

    @pl.when(i != 0)
    def _():
      num_ref[...] = num_ref[...] + nu
      den_ref[...] = den_ref[...] + de

  return pl.pallas_call(
      kfn,
      grid=(grid,),
      in_specs=[
          pl.BlockSpec((blk, 32), lambda i: (i, 0)),
          pl.BlockSpec((1, 32), lambda i: (0, 0)),
          pl.BlockSpec((blk, 1), lambda i: (i, 0)),
          pl.BlockSpec((1, 1), lambda i: (0, 0)),
      ],
      out_specs=[
          pl.BlockSpec((1, 32), lambda i: (0, 0)),
          pl.BlockSpec((1, 1), lambda i: (0, 0)),
      ],
      out_shape=[
          jax.ShapeDtypeStruct((1, 32), jnp.float32),
          jax.ShapeDtypeStruct((1, 1), jnp.float32),
      ],
  )(h, cm, g, gm)


def _final(num1, den1, cs1, num2, den2, cs2, lin_w, lin_b, cls_w, cls_b):
  def kfn(n1, d1, c1, n2, d2, c2, lw, lb, cw, cb, o_ref):
    hg = n1[...] / (c1[...] * d1[...]) + n2[...] / (c2[...] * d2[...])
    hid = jnp.maximum(
        jnp.dot(hg, lw[...], preferred_element_type=jnp.float32) + lb[...],
        0.0)
    o_ref[...] = jnp.dot(hid, cw[...],
                         preferred_element_type=jnp.float32) + cb[...]

  return pl.pallas_call(
      kfn,
      out_shape=jax.ShapeDtypeStruct((1, cls_w.shape[1]), jnp.float32),
  )(num1, den1, cs1, num2, den2, cs2, lin_w, lin_b, cls_w, cls_b)


@jax.jit
def kernel(x, edge_index, W_src0, W_dst0, attn0, bias0, res0, W_src1, W_dst1,
           attn1, bias1, gate_W, gate_b, lin_W, lin_b, cls_W, cls_b):
  src = edge_index[0]
  dst = edge_index[1]
  gw = gate_W.reshape(1, -1)
  b0 = bias0.reshape(1, -1)
  b1 = bias1.reshape(1, -1)
  lb = lin_b.reshape(1, -1)
  cb = cls_b.reshape(1, -1)

  p0 = _mm(x, jnp.concatenate([W_src0, W_dst0, res0], axis=1))
  acc0 = _edge_sc(p0[:, 0:16], p0[:, 16:32], p0[:, 32:48], p0[:, 48:64],
                  src, dst, attn0)
  h1, cm1 = _combine(acc0, p0[:, 64:96], b0)
  cs1 = _colsumexp(h1, cm1)
  g1, gm1 = _gate(h1, cm1, cs1, gw)
  num1, den1 = _poolsum(h1, cm1, g1, gm1)

  p1 = _mm(h1, jnp.concatenate([W_src1, W_dst1], axis=1))
  acc1 = _edge_sc(p1[:, 0:16], p1[:, 16:32], p1[:, 32:48], p1[:, 48:64],
                  src, dst, attn1)
  h2, cm2 = _combine(acc1, h1, b1)
  cs2 = _colsumexp(h2, cm2)
  g2, gm2 = _gate(h2, cm2, cs2, gw)
  num2, den2 = _poolsum(h2, cm2, g2, gm2)

  return _final(num1, den1, cs1, num2, den2, cs2,
                lin_W, lb, cls_W, cb)


# speedup vs baseline: 124.6168x; 1.4665x over previous
import functools

import jax
import jax.numpy as jnp
from jax import lax
from jax.experimental import pallas as pl
from jax.experimental.pallas import tpu as pltpu
from jax.experimental.pallas import tpu_sc as plsc

_NW = 32
_C = 128
_RB = 112


def _edge_sc(fs0, fs1, fd0, fd1, src, dst, attn):
  n = fs0.shape[0]
  e = src.shape[0]
  epw = e // _NW
  nchunks = epw // _C
  tail = epw - nchunks * _C
  rpt = -(-(-(-n // 16)) // _RB) * _RB
  npad = 16 * rpt
  nrb = rpt // _RB

  mesh = plsc.VectorSubcoreMesh(core_axis_name="c", subcore_axis_name="s")

  def body(fs0_h, fs1_h, fd0_h, fd1_h, src_h, dst_h, attn_h, out_h,
           table, srcva, dstva, dsca, fsra, fdra, contriba,
           srcvb, dstvb, dscb, fsrb, fdrb, contribb,
           srcvt, dstvt, fsrt, fdrt, contribt,
           zbuf, tmpb, attnv, semga, semgb, semsa, semsb):
    cid = lax.axis_index("c")
    sid = lax.axis_index("s")
    wid = sid * 2 + cid
    ebase = wid * epw
    tbase = sid * rpt

    ii = lax.iota(jnp.int32, 16)
    onehot = jnp.where(ii == 0, 1.0, 0.0).astype(jnp.float32)
    xors = [ii ^ sh for sh in (1, 2, 4, 8)]

    def _dyng(t, idx):
      dn = lax.GatherDimensionNumbers(
          offset_dims=(), collapsed_slice_dims=(0,), start_index_map=(0,))
      return lax.gather(t, idx[:, None], dn, (1,),
                        mode=lax.GatherScatterMode.PROMISE_IN_BOUNDS)

    def zb(r, c):
      zbuf[r, 0:16] = jnp.zeros((16,), jnp.float32)
      zbuf[r, 16:32] = jnp.zeros((16,), jnp.float32)
      return c
    lax.fori_loop(0, _RB, zb, 0)

    def zero_table():
      for b in range(nrb):
        pltpu.sync_copy(zbuf, table.at[pl.ds(tbase + b * _RB, _RB)])

    def run_phase(h, fs_t, fd_t):
      pltpu.sync_copy(attn_h.at[pl.ds(h * 16, 16)], attnv)
      attn_vec = attnv[...]

      def edge_loop(fr, dr, cb, sz):
        @plsc.parallel_loop(0, sz, unroll=8)
        def _(ei):
          a = fr[ei, 0:16]
          bb = dr[ei, 0:16]
          s = a + bb
          t = jnp.maximum(s, 0.2 * s) * attn_vec
          for ix in xors:
            t = t + _dyng(t, ix)
          exv = jnp.exp(t)
          cb[ei, 0:16] = exv * a
          cb[ei, 16:32] = exv * onehot

      setA = (srcva, dstva, dsca, fsra, fdra, contriba, semga, semsa)
      setB = (srcvb, dstvb, dscb, fsrb, fdrb, contribb, semgb, semsb)

      def issue(ci, bufs):
        sv, dv, _, fr, dr, _, semg, _ = bufs
        base = ebase + ci * _C
        pltpu.sync_copy(src_h.at[pl.ds(base, _C)], sv)
        pltpu.sync_copy(dst_h.at[pl.ds(base, _C)], dv)
        pltpu.async_copy(fs_t.at[sv], fr, semg)
        pltpu.async_copy(fd_t.at[dv], dr, semg)

      def wait_gather(bufs):
        sv, dv, _, fr, dr, _, semg, _ = bufs
        pltpu.make_async_copy(fs_t.at[sv], fr, semg).wait()
        pltpu.make_async_copy(fd_t.at[dv], dr, semg).wait()

      def compute_scatter(bufs):
        _, dv, dsc, fr, dr, cb, _, sems = bufs
        for r in range(_C // 16):
          dsc[pl.ds(r * 16, 16)] = dv[pl.ds(r * 16, 16)]
        edge_loop(fr, dr, cb, _C)
        pltpu.async_copy(cb, table.at[dsc], sems, add=True)

      def wait_scat(bufs):
        _, _, dsc, _, _, cb, _, sems = bufs
        pltpu.make_async_copy(cb, table.at[dsc], sems).wait()

      issue(0, setA)
      issue(1, setB)
      wait_gather(setA)
      compute_scatter(setA)
      issue(2, setA)
      wait_gather(setB)
      compute_scatter(setB)
      issue(3, setB)

      def pair(j, c):
        wait_gather(setA)
        wait_scat(setA)
        compute_scatter(setA)
        issue(2 * j + 2, setA)
        wait_gather(setB)
        wait_scat(setB)
        compute_scatter(setB)
        issue(2 * j + 3, setB)
        return c
      lax.fori_loop(1, nchunks // 2 - 1, pair, 0)

      wait_gather(setA)
      wait_scat(setA)
      compute_scatter(setA)
      wait_gather(setB)
      wait_scat(setB)
      compute_scatter(setB)
      wait_scat(setA)
      wait_scat(setB)

      if tail:
        base = ebase + nchunks * _C
        pltpu.sync_copy(src_h.at[pl.ds(base, tail)], srcvt)
        pltpu.sync_copy(dst_h.at[pl.ds(base, tail)], dstvt)
        pltpu.async_copy(fs_t.at[srcvt], fsrt, semga)
        pltpu.async_copy(fd_t.at[dstvt], fdrt, semgb)
        pltpu.make_async_copy(fs_t.at[srcvt], fsrt, semga).wait()
        pltpu.make_async_copy(fd_t.at[dstvt], fdrt, semgb).wait()
        edge_loop(fsrt, fdrt, contribt, tail)
        pltpu.async_copy(contribt, table.at[dstvt], semsa, add=True)
        pltpu.make_async_copy(contribt, table.at[dstvt], semsa).wait()

    def flush(h):
      for b in range(nrb):
        r0 = tbase + b * _RB
        pltpu.sync_copy(table.at[pl.ds(r0, _RB)], tmpb)
        off = (cid * 2 + h) * npad + r0
        pltpu.sync_copy(tmpb, out_h.at[pl.ds(off, _RB)])

    zero_table()
    plsc.subcore_barrier()
    run_phase(0, fs0_h, fd0_h)
    plsc.subcore_barrier()
    flush(0)
    zero_table()
    plsc.subcore_barrier()
    run_phase(1, fs1_h, fd1_h)
    plsc.subcore_barrier()
    flush(1)

  f = pl.kernel(
      body,
      out_type=jax.ShapeDtypeStruct((4 * npad, 32), jnp.float32),
      mesh=mesh,
      compiler_params=pltpu.CompilerParams(use_tc_tiling_on_sc=False),
      scratch_types=(
          [pltpu.VMEM_SHARED((npad, 32), jnp.float32)]
          + 2 * [
              pltpu.VMEM((_C,), jnp.int32),
              pltpu.VMEM((_C,), jnp.int32),
              pltpu.VMEM((_C,), jnp.int32),
              pltpu.VMEM((_C, 16), jnp.float32),
              pltpu.VMEM((_C, 16), jnp.float32),
              pltpu.VMEM((_C, 32), jnp.float32),
          ]
          + [
              pltpu.VMEM((tail or 8,), jnp.int32),
              pltpu.VMEM((tail or 8,), jnp.int32),
              pltpu.VMEM((tail or 8, 16), jnp.float32),
              pltpu.VMEM((tail or 8, 16), jnp.float32),
              pltpu.VMEM((tail or 8, 32), jnp.float32),
              pltpu.VMEM((_RB, 32), jnp.float32),
              pltpu.VMEM((_RB, 32), jnp.float32),
              pltpu.VMEM((16,), jnp.float32),
              pltpu.SemaphoreType.DMA,
              pltpu.SemaphoreType.DMA,
              pltpu.SemaphoreType.DMA,
              pltpu.SemaphoreType.DMA,
          ]
      ),
  )
  out = f(fs0, fs1, fd0, fd1, src, dst, attn.reshape(-1))
  return out.reshape(2, 2, npad, 32)[:, :, :n, :]


def _mm(x, w):
  n, k = x.shape
  m = w.shape[1]
  blk = 1000
  grid = n // blk
  return pl.pallas_call(
      lambda x_ref, w_ref, o_ref: o_ref.__setitem__(
          (...,), jnp.dot(x_ref[...], w_ref[...],
                          preferred_element_type=jnp.float32)),
      grid=(grid,),
      in_specs=[
          pl.BlockSpec((blk, k), lambda i: (i, 0)),
          pl.BlockSpec((k, m), lambda i: (0, 0)),
      ],
      out_specs=pl.BlockSpec((blk, m), lambda i: (i, 0)),
      out_shape=jax.ShapeDtypeStruct((n, m), jnp.float32),
  )(x, w)


def _combine(acc, rv, bias):
  n = rv.shape[0]
  blk = 1000
  grid = n // blk

  def kfn(acc_ref, rv_ref, b_ref, o_ref, cm_ref):
    i = pl.program_id(0)
    a0 = acc_ref[0, 0] + acc_ref[1, 0]
    a1 = acc_ref[0, 1] + acc_ref[1, 1]
    num = jnp.concatenate([a0[:, 0:16], a1[:, 0:16]], axis=1)
    den = jnp.concatenate([
        jnp.broadcast_to(a0[:, 16:17], (blk, 16)),
        jnp.broadcast_to(a1[:, 16:17], (blk, 16)),
    ], axis=1)
    h = jnp.maximum(num / (den + 1e-16) + rv_ref[...] + b_ref[...], 0.0)
    o_ref[...] = h
    bm = jnp.max(h, axis=0, keepdims=True)

    @pl.when(i == 0)
    def _():
      cm_ref[...] = bm

    @pl.when(i != 0)
    def _():
      cm_ref[...] = jnp.maximum(cm_ref[...], bm)

  return pl.pallas_call(
      kfn,
      grid=(grid,),
      in_specs=[
          pl.BlockSpec((2, 2, blk, 32), lambda i: (0, 0, i, 0)),
          pl.BlockSpec((blk, 32), lambda i: (i, 0)),
          pl.BlockSpec((1, 32), lambda i: (0, 0)),
      ],
      out_specs=[
          pl.BlockSpec((blk, 32), lambda i: (i, 0)),
          pl.BlockSpec((1, 32), lambda i: (0, 0)),
      ],
      out_shape=[
          jax.ShapeDtypeStruct((n, 32), jnp.float32),
          jax.ShapeDtypeStruct((1, 32), jnp.float32),
      ],
  )(acc, rv, bias)


def _colsumexp(h, cm):
  n = h.shape[0]
  blk = 1000
  grid = n // blk

  def kfn(h_ref, cm_ref, cs_ref):
    i = pl.program_id(0)
    s = jnp.sum(jnp.exp(h_ref[...] - cm_ref[...]), axis=0, keepdims=True)

    @pl.when(i == 0)
    def _():
      cs_ref[...] = s

    @pl.when(i != 0)
    def _():
      cs_ref[...] = cs_ref[...] + s

  return pl.pallas_call(
      kfn,
      grid=(grid,),
      in_specs=[
          pl.BlockSpec((blk, 32), lambda i: (i, 0)),
          pl.BlockSpec((1, 32), lambda i: (0, 0)),
      ],
      out_specs=pl.BlockSpec((1, 32), lambda i: (0, 0)),
      out_shape=jax.ShapeDtypeStruct((1, 32), jnp.float32),
  )(h, cm)


def _gate(h, cm, cs, gw):
  n = h.shape[0]
  blk = 1000
  grid = n // blk

  def kfn(h_ref, cm_ref, cs_ref, gw_ref, g_ref, gm_ref):
    i = pl.program_id(0)
    w = gw_ref[...] / cs_ref[...]
    g = jnp.sum(jnp.exp(h_ref[...] - cm_ref[...]) * w, axis=1, keepdims=True)
    g_ref[...] = g
    bm = jnp.max(g, axis=0, keepdims=True)[:, 0:1]

    @pl.when(i == 0)
    def _():
      gm_ref[...] = bm

    @pl.when(i != 0)
    def _():
      gm_ref[...] = jnp.maximum(gm_ref[...], bm)

  return pl.pallas_call(
      kfn,
      grid=(grid,),
      in_specs=[
          pl.BlockSpec((blk, 32), lambda i: (i, 0)),
          pl.BlockSpec((1, 32), lambda i: (0, 0)),
          pl.BlockSpec((1, 32), lambda i: (0, 0)),
          pl.BlockSpec((1, 32), lambda i: (0, 0)),
      ],
      out_specs=[
          pl.BlockSpec((blk, 1), lambda i: (i, 0)),
          pl.BlockSpec((1, 1), lambda i: (0, 0)),
      ],
      out_shape=[
          jax.ShapeDtypeStruct((n, 1), jnp.float32),
          jax.ShapeDtypeStruct((1, 1), jnp.float32),
      ],
  )(h, cm, cs, gw)


def _poolsum(h, cm, g, gm):
  n = h.shape[0]
  blk = 1000
  grid = n // blk

  def kfn(h_ref, cm_ref, g_ref, gm_ref, num_ref, den_ref):
    i = pl.program_id(0)
    eg = jnp.exp(g_ref[...] - gm_ref[...])
    nu = jnp.sum(eg * jnp.exp(h_ref[...] - cm_ref[...]), axis=0, keepdims=True)
    de = jnp.sum(eg, axis=0, keepdims=True)

    @pl.when(i == 0)
    def _():
      num_ref[...] = nu
      den_ref[...] = de

    @pl.when(i != 0)
    def _():
      num_ref[...] = num_ref[...] + nu
      den_ref[...] = den_ref[...] + de

  return pl.pallas_call(
      kfn,
      grid=(grid,),
      in_specs=[
          pl.BlockSpec((blk, 32), lambda i: (i, 0)),
          pl.BlockSpec((1, 32), lambda i: (0, 0)),
          pl.BlockSpec((blk, 1), lambda i: (i, 0)),
          pl.BlockSpec((1, 1), lambda i: (0, 0)),
      ],
      out_specs=[
          pl.BlockSpec((1, 32), lambda i: (0, 0)),
          pl.BlockSpec((1, 1), lambda i: (0, 0)),
      ],
      out_shape=[
          jax.ShapeDtypeStruct((1, 32), jnp.float32),
          jax.ShapeDtypeStruct((1, 1), jnp.float32),
      ],
  )(h, cm, g, gm)


def _final(num1, den1, cs1, num2, den2, cs2, lin_w, lin_b, cls_w, cls_b):
  def kfn(n1, d1, c1, n2, d2, c2, lw, lb, cw, cb, o_ref):
    hg = n1[...] / (c1[...] * d1[...]) + n2[...] / (c2[...] * d2[...])
    hid = jnp.maximum(
        jnp.dot(hg, lw[...], preferred_element_type=jnp.float32) + lb[...],
        0.0)
    o_ref[...] = jnp.dot(hid, cw[...],
                         preferred_element_type=jnp.float32) + cb[...]

  return pl.pallas_call(
      kfn,
      out_shape=jax.ShapeDtypeStruct((1, cls_w.shape[1]), jnp.float32),
  )(num1, den1, cs1, num2, den2, cs2, lin_w, lin_b, cls_w, cls_b)


@jax.jit
def kernel(x, edge_index, W_src0, W_dst0, attn0, bias0, res0, W_src1, W_dst1,
           attn1, bias1, gate_W, gate_b, lin_W, lin_b, cls_W, cls_b):
  src = edge_index[0]
  dst = edge_index[1]
  gw = gate_W.reshape(1, -1)
  b0 = bias0.reshape(1, -1)
  b1 = bias1.reshape(1, -1)
  lb = lin_b.reshape(1, -1)
  cb = cls_b.reshape(1, -1)

  p0 = _mm(x, jnp.concatenate([W_src0, W_dst0, res0], axis=1))
  acc0 = _edge_sc(p0[:, 0:16], p0[:, 16:32], p0[:, 32:48], p0[:, 48:64],
                  src, dst, attn0)
  h1, cm1 = _combine(acc0, p0[:, 64:96], b0)
  cs1 = _colsumexp(h1, cm1)
  g1, gm1 = _gate(h1, cm1, cs1, gw)
  num1, den1 = _poolsum(h1, cm1, g1, gm1)

  p1 = _mm(h1, jnp.concatenate([W_src1, W_dst1], axis=1))
  acc1 = _edge_sc(p1[:, 0:16], p1[:, 16:32], p1[:, 32:48], p1[:, 48:64],
                  src, dst, attn1)
  h2, cm2 = _combine(acc1, h1, b1)
  cs2 = _colsumexp(h2, cm2)
  g2, gm2 = _gate(h2, cm2, cs2, gw)
  num2, den2 = _poolsum(h2, cm2, g2, gm2)

  return _final(num1, den1, cs1, num2, den2, cs2,
                lin_W, lb, cls_W, cb)


# speedup vs baseline: 127.4094x; 1.0224x over previous
import functools

import jax
import jax.numpy as jnp
from jax import lax
from jax.experimental import pallas as pl
from jax.experimental.pallas import tpu as pltpu
from jax.experimental.pallas import tpu_sc as plsc

_NW = 32
_C = 128
_RB = 112


def _edge_sc(fs0, fs1, fd0, fd1, src, dst, attn):
  n = fs0.shape[0]
  e = src.shape[0]
  epw = e // _NW
  nchunks = epw // _C
  tail = epw - nchunks * _C
  rpt = -(-(-(-n // 16)) // _RB) * _RB
  npad = 16 * rpt
  nrb = rpt // _RB

  mesh = plsc.VectorSubcoreMesh(core_axis_name="c", subcore_axis_name="s")

  def body(fs0_h, fs1_h, fd0_h, fd1_h, src_h, dst_h, attn_h, out_h,
           table, srcva, dstva, dsca, fsra, fdra, contriba,
           srcvb, dstvb, dscb, fsrb, fdrb, contribb,
           srcvt, dstvt, fsrt, fdrt, contribt,
           zbuf, tmpb, attnv, semga, semgb, semsa, semsb):
    cid = lax.axis_index("c")
    sid = lax.axis_index("s")
    wid = sid * 2 + cid
    ebase = wid * epw
    tbase = sid * rpt

    ii = lax.iota(jnp.int32, 16)
    onehot = jnp.where(ii == 0, 1.0, 0.0).astype(jnp.float32)
    xors = [ii ^ sh for sh in (1, 2, 4, 8)]

    def _dyng(t, idx):
      dn = lax.GatherDimensionNumbers(
          offset_dims=(), collapsed_slice_dims=(0,), start_index_map=(0,))
      return lax.gather(t, idx[:, None], dn, (1,),
                        mode=lax.GatherScatterMode.PROMISE_IN_BOUNDS)

    def zb(r, c):
      zbuf[r, 0:16] = jnp.zeros((16,), jnp.float32)
      zbuf[r, 16:32] = jnp.zeros((16,), jnp.float32)
      return c
    lax.fori_loop(0, _RB, zb, 0)

    def zero_table():
      for b in range(nrb):
        pltpu.sync_copy(zbuf, table.at[pl.ds(tbase + b * _RB, _RB)])

    def run_phase(h, fs_t, fd_t):
      pltpu.sync_copy(attn_h.at[pl.ds(h * 16, 16)], attnv)
      attn_vec = attnv[...]

      def edge_loop(fr, dr, cb, sz):
        @plsc.parallel_loop(0, sz, unroll=8)
        def _(ei):
          a = fr[ei, 0:16]
          bb = dr[ei, 0:16]
          s = a + bb
          t = jnp.maximum(s, 0.2 * s) * attn_vec
          for ix in xors:
            t = t + _dyng(t, ix)
          exv = jnp.exp(t)
          cb[ei, 0:16] = exv * a
          cb[ei, 16:32] = exv * onehot

      setA = (srcva, dstva, dsca, fsra, fdra, contriba, semga, semsa)
      setB = (srcvb, dstvb, dscb, fsrb, fdrb, contribb, semgb, semsb)

      def issue(ci, bufs):
        sv, dv, _, fr, dr, _, semg, _ = bufs
        base = ebase + ci * _C
        pltpu.sync_copy(src_h.at[pl.ds(base, _C)], sv)
        pltpu.sync_copy(dst_h.at[pl.ds(base, _C)], dv)
        pltpu.async_copy(fs_t.at[sv], fr, semg)
        pltpu.async_copy(fd_t.at[dv], dr, semg)

      def wait_gather(bufs):
        sv, dv, _, fr, dr, _, semg, _ = bufs
        pltpu.make_async_copy(fs_t.at[sv], fr, semg).wait()
        pltpu.make_async_copy(fd_t.at[dv], dr, semg).wait()

      def compute_scatter(bufs):
        _, dv, dsc, fr, dr, cb, _, sems = bufs
        for r in range(_C // 16):
          dsc[pl.ds(r * 16, 16)] = dv[pl.ds(r * 16, 16)]
        edge_loop(fr, dr, cb, _C)
        pltpu.async_copy(cb, table.at[dsc], sems, add=True)

      def wait_scat(bufs):
        _, _, dsc, _, _, cb, _, sems = bufs
        pltpu.make_async_copy(cb, table.at[dsc], sems).wait()

      issue(0, setA)
      issue(1, setB)
      wait_gather(setA)
      compute_scatter(setA)
      issue(2, setA)
      wait_gather(setB)
      compute_scatter(setB)
      issue(3, setB)

      def pair(j, c):
        wait_gather(setA)
        wait_scat(setA)
        compute_scatter(setA)
        issue(2 * j + 2, setA)
        wait_gather(setB)
        wait_scat(setB)
        compute_scatter(setB)
        issue(2 * j + 3, setB)
        return c
      lax.fori_loop(1, nchunks // 2 - 1, pair, 0)

      wait_gather(setA)
      wait_scat(setA)
      compute_scatter(setA)
      wait_gather(setB)
      wait_scat(setB)
      compute_scatter(setB)
      wait_scat(setA)
      wait_scat(setB)

      if tail:
        base = ebase + nchunks * _C
        pltpu.sync_copy(src_h.at[pl.ds(base, tail)], srcvt)
        pltpu.sync_copy(dst_h.at[pl.ds(base, tail)], dstvt)
        pltpu.async_copy(fs_t.at[srcvt], fsrt, semga)
        pltpu.async_copy(fd_t.at[dstvt], fdrt, semgb)
        pltpu.make_async_copy(fs_t.at[srcvt], fsrt, semga).wait()
        pltpu.make_async_copy(fd_t.at[dstvt], fdrt, semgb).wait()
        edge_loop(fsrt, fdrt, contribt, tail)
        pltpu.async_copy(contribt, table.at[dstvt], semsa, add=True)
        pltpu.make_async_copy(contribt, table.at[dstvt], semsa).wait()

    def flush(h):
      for b in range(nrb):
        r0 = tbase + b * _RB
        pltpu.sync_copy(table.at[pl.ds(r0, _RB)], tmpb)
        off = (cid * 2 + h) * npad + r0
        pltpu.sync_copy(tmpb, out_h.at[pl.ds(off, _RB)])

    zero_table()
    plsc.subcore_barrier()
    run_phase(0, fs0_h, fd0_h)
    plsc.subcore_barrier()
    flush(0)
    zero_table()
    plsc.subcore_barrier()
    run_phase(1, fs1_h, fd1_h)
    plsc.subcore_barrier()
    flush(1)

  f = pl.kernel(
      body,
      out_type=jax.ShapeDtypeStruct((4 * npad, 32), jnp.float32),
      mesh=mesh,
      compiler_params=pltpu.CompilerParams(use_tc_tiling_on_sc=False),
      scratch_types=(
          [pltpu.VMEM_SHARED((npad, 32), jnp.float32)]
          + 2 * [
              pltpu.VMEM((_C,), jnp.int32),
              pltpu.VMEM((_C,), jnp.int32),
              pltpu.VMEM((_C,), jnp.int32),
              pltpu.VMEM((_C, 16), jnp.float32),
              pltpu.VMEM((_C, 16), jnp.float32),
              pltpu.VMEM((_C, 32), jnp.float32),
          ]
          + [
              pltpu.VMEM((tail or 8,), jnp.int32),
              pltpu.VMEM((tail or 8,), jnp.int32),
              pltpu.VMEM((tail or 8, 16), jnp.float32),
              pltpu.VMEM((tail or 8, 16), jnp.float32),
              pltpu.VMEM((tail or 8, 32), jnp.float32),
              pltpu.VMEM((_RB, 32), jnp.float32),
              pltpu.VMEM((_RB, 32), jnp.float32),
              pltpu.VMEM((16,), jnp.float32),
              pltpu.SemaphoreType.DMA,
              pltpu.SemaphoreType.DMA,
              pltpu.SemaphoreType.DMA,
              pltpu.SemaphoreType.DMA,
          ]
      ),
  )
  out = f(fs0, fs1, fd0, fd1, src, dst, attn.reshape(-1))
  return out.reshape(2, 2, npad, 32)[:, :, :n, :]


def _mm(x, w, with_res):
  n, k = x.shape
  m = w.shape[1]
  blk = 1000
  grid = n // blk

  def kfn(x_ref, w_ref, *outs):
    p = jnp.dot(x_ref[...], w_ref[...], preferred_element_type=jnp.float32)
    for q in range(4):
      outs[q][...] = p[:, 16 * q:16 * (q + 1)]
    if with_res:
      outs[4][...] = p[:, 64:96]

  nouts = 5 if with_res else 4
  out_shape = [jax.ShapeDtypeStruct((n, 16), jnp.float32)] * 4
  out_specs = [pl.BlockSpec((blk, 16), lambda i: (i, 0))] * 4
  if with_res:
    out_shape.append(jax.ShapeDtypeStruct((n, 32), jnp.float32))
    out_specs.append(pl.BlockSpec((blk, 32), lambda i: (i, 0)))
  return pl.pallas_call(
      kfn,
      grid=(grid,),
      in_specs=[
          pl.BlockSpec((blk, k), lambda i: (i, 0)),
          pl.BlockSpec((k, m), lambda i: (0, 0)),
      ],
      out_specs=out_specs,
      out_shape=out_shape,
  )(x, w)


def _combine(acc, rv, bias):
  n = rv.shape[0]
  blk = 1000
  grid = n // blk

  def kfn(acc_ref, rv_ref, b_ref, o_ref, cm_ref):
    i = pl.program_id(0)
    a0 = acc_ref[0, 0] + acc_ref[1, 0]
    a1 = acc_ref[0, 1] + acc_ref[1, 1]
    num = jnp.concatenate([a0[:, 0:16], a1[:, 0:16]], axis=1)
    den = jnp.concatenate([
        jnp.broadcast_to(a0[:, 16:17], (blk, 16)),
        jnp.broadcast_to(a1[:, 16:17], (blk, 16)),
    ], axis=1)
    h = jnp.maximum(num / (den + 1e-16) + rv_ref[...] + b_ref[...], 0.0)
    o_ref[...] = h
    bm = jnp.max(h, axis=0, keepdims=True)

    @pl.when(i == 0)
    def _():
      cm_ref[...] = bm

    @pl.when(i != 0)
    def _():
      cm_ref[...] = jnp.maximum(cm_ref[...], bm)

  return pl.pallas_call(
      kfn,
      grid=(grid,),
      in_specs=[
          pl.BlockSpec((2, 2, blk, 32), lambda i: (0, 0, i, 0)),
          pl.BlockSpec((blk, 32), lambda i: (i, 0)),
          pl.BlockSpec((1, 32), lambda i: (0, 0)),
      ],
      out_specs=[
          pl.BlockSpec((blk, 32), lambda i: (i, 0)),
          pl.BlockSpec((1, 32), lambda i: (0, 0)),
      ],
      out_shape=[
          jax.ShapeDtypeStruct((n, 32), jnp.float32),
          jax.ShapeDtypeStruct((1, 32), jnp.float32),
      ],
  )(acc, rv, bias)


def _pool(h, cm, gw):
  n = h.shape[0]
  blk = 1000
  grid = n // blk

  def kfn(h_ref, cm_ref, gw_ref, cs_ref, g_ref, gm_ref, num_ref, den_ref):
    p = pl.program_id(0)
    i = pl.program_id(1)
    eh = jnp.exp(h_ref[...] - cm_ref[...])

    @pl.when(p == 0)
    def _():
      s = jnp.sum(eh, axis=0, keepdims=True)

      @pl.when(i == 0)
      def _():
        cs_ref[...] = s

      @pl.when(i != 0)
      def _():
        cs_ref[...] = cs_ref[...] + s

    @pl.when(p == 1)
    def _():
      g = jnp.sum(eh * (gw_ref[...] / cs_ref[...]), axis=1, keepdims=True)
      g_ref[...] = g
      bm = jnp.max(g, axis=0, keepdims=True)[:, 0:1]

      @pl.when(i == 0)
      def _():
        gm_ref[...] = bm

      @pl.when(i != 0)
      def _():
        gm_ref[...] = jnp.maximum(gm_ref[...], bm)

    @pl.when(p == 2)
    def _():
      eg = jnp.exp(g_ref[...] - gm_ref[...])
      nu = jnp.sum(eg * eh, axis=0, keepdims=True)
      de = jnp.sum(eg, axis=0, keepdims=True)

      @pl.when(i == 0)
      def _():
        num_ref[...] = nu
        den_ref[...] = de

      @pl.when(i != 0)
      def _():
        num_ref[...] = num_ref[...] + nu
        den_ref[...] = den_ref[...] + de

  cs, _, _, num, den = pl.pallas_call(
      kfn,
      grid=(3, grid),
      in_specs=[
          pl.BlockSpec((blk, 32), lambda p, i: (i, 0)),
          pl.BlockSpec((1, 32), lambda p, i: (0, 0)),
          pl.BlockSpec((1, 32), lambda p, i: (0, 0)),
      ],
      out_specs=[
          pl.BlockSpec((1, 32), lambda p, i: (0, 0)),
          pl.BlockSpec((blk, 1), lambda p, i: (i, 0)),
          pl.BlockSpec((1, 1), lambda p, i: (0, 0)),
          pl.BlockSpec((1, 32), lambda p, i: (0, 0)),
          pl.BlockSpec((1, 1), lambda p, i: (0, 0)),
      ],
      out_shape=[
          jax.ShapeDtypeStruct((1, 32), jnp.float32),
          jax.ShapeDtypeStruct((n, 1), jnp.float32),
          jax.ShapeDtypeStruct((1, 1), jnp.float32),
          jax.ShapeDtypeStruct((1, 32), jnp.float32),
          jax.ShapeDtypeStruct((1, 1), jnp.float32),
      ],
  )(h, cm, gw)
  return cs, num, den


def _final(num1, den1, cs1, num2, den2, cs2, lin_w, lin_b, cls_w, cls_b):
  def kfn(n1, d1, c1, n2, d2, c2, lw, lb, cw, cb, o_ref):
    hg = n1[...] / (c1[...] * d1[...]) + n2[...] / (c2[...] * d2[...])
    hid = jnp.maximum(
        jnp.dot(hg, lw[...], preferred_element_type=jnp.float32) + lb[...],
        0.0)
    o_ref[...] = jnp.dot(hid, cw[...],
                         preferred_element_type=jnp.float32) + cb[...]

  return pl.pallas_call(
      kfn,
      out_shape=jax.ShapeDtypeStruct((1, cls_w.shape[1]), jnp.float32),
  )(num1, den1, cs1, num2, den2, cs2, lin_w, lin_b, cls_w, cls_b)


@jax.jit
def kernel(x, edge_index, W_src0, W_dst0, attn0, bias0, res0, W_src1, W_dst1,
           attn1, bias1, gate_W, gate_b, lin_W, lin_b, cls_W, cls_b):
  src = edge_index[0]
  dst = edge_index[1]
  gw = gate_W.reshape(1, -1)
  b0 = bias0.reshape(1, -1)
  b1 = bias1.reshape(1, -1)
  lb = lin_b.reshape(1, -1)
  cb = cls_b.reshape(1, -1)

  fs00, fs01, fd00, fd01, rv0 = _mm(
      x, jnp.concatenate([W_src0, W_dst0, res0], axis=1), True)
  acc0 = _edge_sc(fs00, fs01, fd00, fd01, src, dst, attn0)
  h1, cm1 = _combine(acc0, rv0, b0)
  cs1, num1, den1 = _pool(h1, cm1, gw)

  fs10, fs11, fd10, fd11 = _mm(
      h1, jnp.concatenate([W_src1, W_dst1], axis=1), False)
  acc1 = _edge_sc(fs10, fs11, fd10, fd11, src, dst, attn1)
  h2, cm2 = _combine(acc1, h1, b1)
  cs2, num2, den2 = _pool(h2, cm2, gw)

  return _final(num1, den1, cs1, num2, den2, cs2,
                lin_W, lb, cls_W, cb)


# speedup vs baseline: 130.1238x; 1.0213x over previous
import functools

import jax
import jax.numpy as jnp
from jax import lax
from jax.experimental import pallas as pl
from jax.experimental.pallas import tpu as pltpu
from jax.experimental.pallas import tpu_sc as plsc

_NW = 32
_C = 128
_RB = 112


def _edge_sc(fs0, fs1, fd0, fd1, src, dst, attn):
  n = fs0.shape[0]
  e = src.shape[0]
  epw = e // _NW
  nchunks = epw // _C
  tail = epw - nchunks * _C
  rpt = -(-(-(-n // 16)) // _RB) * _RB
  npad = 16 * rpt
  nrb = rpt // _RB

  mesh = plsc.VectorSubcoreMesh(core_axis_name="c", subcore_axis_name="s")

  def body(fs0_h, fs1_h, fd0_h, fd1_h, src_h, dst_h, attn_h, out_h,
           table, srcva, dstva, dsca, fsra, fdra, contriba,
           srcvb, dstvb, dscb, fsrb, fdrb, contribb,
           srcvt, dstvt, fsrt, fdrt, contribt,
           zbuf, tmpb, attnv, semga, semgb, semsa, semsb):
    cid = lax.axis_index("c")
    sid = lax.axis_index("s")
    wid = sid * 2 + cid
    ebase = wid * epw
    tbase = sid * rpt

    ii = lax.iota(jnp.int32, 16)
    xors = [ii ^ sh for sh in (1, 2, 4, 8)]

    def _dyng(t, idx):
      dn = lax.GatherDimensionNumbers(
          offset_dims=(), collapsed_slice_dims=(0,), start_index_map=(0,))
      return lax.gather(t, idx[:, None], dn, (1,),
                        mode=lax.GatherScatterMode.PROMISE_IN_BOUNDS)

    def zb(r, c):
      zbuf[r, 0:16] = jnp.zeros((16,), jnp.float32)
      zbuf[r, 16:32] = jnp.zeros((16,), jnp.float32)
      return c
    lax.fori_loop(0, _RB, zb, 0)

    def zero_table():
      for b in range(nrb):
        pltpu.sync_copy(zbuf, table.at[pl.ds(tbase + b * _RB, _RB)])

    def run_phase(h, fs_t, fd_t):
      pltpu.sync_copy(attn_h.at[pl.ds(h * 16, 16)], attnv)
      attn_vec = attnv[...]

      def edge_loop(fr, dr, cb, sz):
        @plsc.parallel_loop(0, sz, unroll=8)
        def _(ei):
          a = fr[ei, 0:16]
          bb = dr[ei, 0:16]
          s = a + bb
          t = jnp.maximum(s, 0.2 * s) * attn_vec
          for ix in xors:
            t = t + _dyng(t, ix)
          exv = jnp.exp(t)
          cb[ei, 0:16] = exv * a
          cb[ei, 16:32] = exv

      setA = (srcva, dstva, dsca, fsra, fdra, contriba, semga, semsa)
      setB = (srcvb, dstvb, dscb, fsrb, fdrb, contribb, semgb, semsb)

      def issue(ci, bufs):
        sv, dv, _, fr, dr, _, semg, _ = bufs
        base = ebase + ci * _C
        pltpu.sync_copy(src_h.at[pl.ds(base, _C)], sv)
        pltpu.sync_copy(dst_h.at[pl.ds(base, _C)], dv)
        pltpu.async_copy(fs_t.at[sv], fr, semg)
        pltpu.async_copy(fd_t.at[dv], dr, semg)

      def wait_gather(bufs):
        sv, dv, _, fr, dr, _, semg, _ = bufs
        pltpu.make_async_copy(fs_t.at[sv], fr, semg).wait()
        pltpu.make_async_copy(fd_t.at[dv], dr, semg).wait()

      def compute_scatter(bufs):
        _, dv, dsc, fr, dr, cb, _, sems = bufs
        for r in range(_C // 16):
          dsc[pl.ds(r * 16, 16)] = dv[pl.ds(r * 16, 16)]
        edge_loop(fr, dr, cb, _C)
        pltpu.async_copy(cb, table.at[dsc], sems, add=True)

      def wait_scat(bufs):
        _, _, dsc, _, _, cb, _, sems = bufs
        pltpu.make_async_copy(cb, table.at[dsc], sems).wait()

      issue(0, setA)
      issue(1, setB)
      wait_gather(setA)
      compute_scatter(setA)
      issue(2, setA)
      wait_gather(setB)
      compute_scatter(setB)
      issue(3, setB)

      def pair(j, c):
        wait_gather(setA)
        wait_scat(setA)
        compute_scatter(setA)
        issue(2 * j + 2, setA)
        wait_gather(setB)
        wait_scat(setB)
        compute_scatter(setB)
        issue(2 * j + 3, setB)
        return c
      lax.fori_loop(1, nchunks // 2 - 1, pair, 0)

      wait_gather(setA)
      wait_scat(setA)
      compute_scatter(setA)
      wait_gather(setB)
      wait_scat(setB)
      compute_scatter(setB)
      wait_scat(setA)
      wait_scat(setB)

      if tail:
        base = ebase + nchunks * _C
        pltpu.sync_copy(src_h.at[pl.ds(base, tail)], srcvt)
        pltpu.sync_copy(dst_h.at[pl.ds(base, tail)], dstvt)
        pltpu.async_copy(fs_t.at[srcvt], fsrt, semga)
        pltpu.async_copy(fd_t.at[dstvt], fdrt, semgb)
        pltpu.make_async_copy(fs_t.at[srcvt], fsrt, semga).wait()
        pltpu.make_async_copy(fd_t.at[dstvt], fdrt, semgb).wait()
        edge_loop(fsrt, fdrt, contribt, tail)
        pltpu.async_copy(contribt, table.at[dstvt], semsa, add=True)
        pltpu.make_async_copy(contribt, table.at[dstvt], semsa).wait()

    def flush(h, rezero):
      bufs = (contriba.at[pl.ds(0, _RB)], contribb.at[pl.ds(0, _RB)], tmpb)
      sems = (semga, semgb, semsa)
      obase = (cid * 2 + h) * npad + tbase

      def rd(b):
        pltpu.async_copy(table.at[pl.ds(tbase + b * _RB, _RB)],
                         bufs[b % 3], sems[b % 3])

      def fin_rd(b):
        pltpu.make_async_copy(table.at[pl.ds(tbase + b * _RB, _RB)],
                              bufs[b % 3], sems[b % 3]).wait()

      def wr(b):
        pltpu.async_copy(bufs[b % 3], out_h.at[pl.ds(obase + b * _RB, _RB)],
                         sems[b % 3])

      def fin_wr(b):
        pltpu.make_async_copy(bufs[b % 3],
                              out_h.at[pl.ds(obase + b * _RB, _RB)],
                              sems[b % 3]).wait()

      rd(0)
      rd(1)
      rd(2)
      for b in range(nrb):
        fin_rd(b)
        wr(b)
        if rezero:
          pltpu.sync_copy(zbuf, table.at[pl.ds(tbase + b * _RB, _RB)])
        if b + 3 < nrb:
          fin_wr(b)
          rd(b + 3)
      for b in range(max(nrb - 3, 0), nrb):
        fin_wr(b)

    zero_table()
    plsc.subcore_barrier()
    run_phase(0, fs0_h, fd0_h)
    plsc.subcore_barrier()
    flush(0, True)
    plsc.subcore_barrier()
    run_phase(1, fs1_h, fd1_h)
    plsc.subcore_barrier()
    flush(1, False)

  f = pl.kernel(
      body,
      out_type=jax.ShapeDtypeStruct((4 * npad, 32), jnp.float32),
      mesh=mesh,
      compiler_params=pltpu.CompilerParams(use_tc_tiling_on_sc=False),
      scratch_types=(
          [pltpu.VMEM_SHARED((npad, 32), jnp.float32)]
          + 2 * [
              pltpu.VMEM((_C,), jnp.int32),
              pltpu.VMEM((_C,), jnp.int32),
              pltpu.VMEM((_C,), jnp.int32),
              pltpu.VMEM((_C, 16), jnp.float32),
              pltpu.VMEM((_C, 16), jnp.float32),
              pltpu.VMEM((_C, 32), jnp.float32),
          ]
          + [
              pltpu.VMEM((tail or 8,), jnp.int32),
              pltpu.VMEM((tail or 8,), jnp.int32),
              pltpu.VMEM((tail or 8, 16), jnp.float32),
              pltpu.VMEM((tail or 8, 16), jnp.float32),
              pltpu.VMEM((tail or 8, 32), jnp.float32),
              pltpu.VMEM((_RB, 32), jnp.float32),
              pltpu.VMEM((_RB, 32), jnp.float32),
              pltpu.VMEM((16,), jnp.float32),
              pltpu.SemaphoreType.DMA,
              pltpu.SemaphoreType.DMA,
              pltpu.SemaphoreType.DMA,
              pltpu.SemaphoreType.DMA,
          ]
      ),
  )
  out = f(fs0, fs1, fd0, fd1, src, dst, attn.reshape(-1))
  return out.reshape(2, 2, npad, 32)[:, :, :n, :]


def _mm(x, w, with_res):
  n, k = x.shape
  m = w.shape[1]
  blk = 1000
  grid = n // blk

  def kfn(x_ref, w_ref, *outs):
    p = jnp.dot(x_ref[...], w_ref[...], preferred_element_type=jnp.float32)
    for q in range(4):
      outs[q][...] = p[:, 16 * q:16 * (q + 1)]
    if with_res:
      outs[4][...] = p[:, 64:96]

  nouts = 5 if with_res else 4
  out_shape = [jax.ShapeDtypeStruct((n, 16), jnp.float32)] * 4
  out_specs = [pl.BlockSpec((blk, 16), lambda i: (i, 0))] * 4
  if with_res:
    out_shape.append(jax.ShapeDtypeStruct((n, 32), jnp.float32))
    out_specs.append(pl.BlockSpec((blk, 32), lambda i: (i, 0)))
  return pl.pallas_call(
      kfn,
      grid=(grid,),
      in_specs=[
          pl.BlockSpec((blk, k), lambda i: (i, 0)),
          pl.BlockSpec((k, m), lambda i: (0, 0)),
      ],
      out_specs=out_specs,
      out_shape=out_shape,
  )(x, w)


def _combine(acc, rv, bias):
  n = rv.shape[0]
  blk = 1000
  grid = n // blk

  def kfn(acc_ref, rv_ref, b_ref, o_ref, cm_ref):
    i = pl.program_id(0)
    a0 = acc_ref[0, 0] + acc_ref[1, 0]
    a1 = acc_ref[0, 1] + acc_ref[1, 1]
    num = jnp.concatenate([a0[:, 0:16], a1[:, 0:16]], axis=1)
    den = jnp.concatenate([
        jnp.broadcast_to(a0[:, 16:17], (blk, 16)),
        jnp.broadcast_to(a1[:, 16:17], (blk, 16)),
    ], axis=1)
    h = jnp.maximum(num / (den + 1e-16) + rv_ref[...] + b_ref[...], 0.0)
    o_ref[...] = h
    bm = jnp.max(h, axis=0, keepdims=True)

    @pl.when(i == 0)
    def _():
      cm_ref[...] = bm

    @pl.when(i != 0)
    def _():
      cm_ref[...] = jnp.maximum(cm_ref[...], bm)

  return pl.pallas_call(
      kfn,
      grid=(grid,),
      in_specs=[
          pl.BlockSpec((2, 2, blk, 32), lambda i: (0, 0, i, 0)),
          pl.BlockSpec((blk, 32), lambda i: (i, 0)),
          pl.BlockSpec((1, 32), lambda i: (0, 0)),
      ],
      out_specs=[
          pl.BlockSpec((blk, 32), lambda i: (i, 0)),
          pl.BlockSpec((1, 32), lambda i: (0, 0)),
      ],
      out_shape=[
          jax.ShapeDtypeStruct((n, 32), jnp.float32),
          jax.ShapeDtypeStruct((1, 32), jnp.float32),
      ],
  )(acc, rv, bias)


def _pool(h, cm, gw):
  n = h.shape[0]
  blk = 1000
  grid = n // blk

  def kfn(h_ref, cm_ref, gw_ref, cs_ref, g_ref, gm_ref, num_ref, den_ref):
    p = pl.program_id(0)
    i = pl.program_id(1)
    eh = jnp.exp(h_ref[...] - cm_ref[...])

    @pl.when(p == 0)
    def _():
      s = jnp.sum(eh, axis=0, keepdims=True)

      @pl.when(i == 0)
      def _():
        cs_ref[...] = s

      @pl.when(i != 0)
      def _():
        cs_ref[...] = cs_ref[...] + s

    @pl.when(p == 1)
    def _():
      g = jnp.sum(eh * (gw_ref[...] / cs_ref[...]), axis=1, keepdims=True)
      g_ref[...] = g
      bm = jnp.max(g, axis=0, keepdims=True)[:, 0:1]

      @pl.when(i == 0)
      def _():
        gm_ref[...] = bm

      @pl.when(i != 0)
      def _():
        gm_ref[...] = jnp.maximum(gm_ref[...], bm)

    @pl.when(p == 2)
    def _():
      eg = jnp.exp(g_ref[...] - gm_ref[...])
      nu = jnp.sum(eg * eh, axis=0, keepdims=True)
      de = jnp.sum(eg, axis=0, keepdims=True)

      @pl.when(i == 0)
      def _():
        num_ref[...] = nu
        den_ref[...] = de

      @pl.when(i != 0)
      def _():
        num_ref[...] = num_ref[...] + nu
        den_ref[...] = den_ref[...] + de

  cs, _, _, num, den = pl.pallas_call(
      kfn,
      grid=(3, grid),
      in_specs=[
          pl.BlockSpec((blk, 32), lambda p, i: (i, 0)),
          pl.BlockSpec((1, 32), lambda p, i: (0, 0)),
          pl.BlockSpec((1, 32), lambda p, i: (0, 0)),
      ],
      out_specs=[
          pl.BlockSpec((1, 32), lambda p, i: (0, 0)),
          pl.BlockSpec((blk, 1), lambda p, i: (i, 0)),
          pl.BlockSpec((1, 1), lambda p, i: (0, 0)),
          pl.BlockSpec((1, 32), lambda p, i: (0, 0)),
          pl.BlockSpec((1, 1), lambda p, i: (0, 0)),
      ],
      out_shape=[
          jax.ShapeDtypeStruct((1, 32), jnp.float32),
          jax.ShapeDtypeStruct((n, 1), jnp.float32),
          jax.ShapeDtypeStruct((1, 1), jnp.float32),
          jax.ShapeDtypeStruct((1, 32), jnp.float32),
          jax.ShapeDtypeStruct((1, 1), jnp.float32),
      ],
  )(h, cm, gw)
  return cs, num, den


def _final(num1, den1, cs1, num2, den2, cs2, lin_w, lin_b, cls_w, cls_b):
  def kfn(n1, d1, c1, n2, d2, c2, lw, lb, cw, cb, o_ref):
    hg = n1[...] / (c1[...] * d1[...]) + n2[...] / (c2[...] * d2[...])
    hid = jnp.maximum(
        jnp.dot(hg, lw[...], preferred_element_type=jnp.float32) + lb[...],
        0.0)
    o_ref[...] = jnp.dot(hid, cw[...],
                         preferred_element_type=jnp.float32) + cb[...]

  return pl.pallas_call(
      kfn,
      out_shape=jax.ShapeDtypeStruct((1, cls_w.shape[1]), jnp.float32),
  )(num1, den1, cs1, num2, den2, cs2, lin_w, lin_b, cls_w, cls_b)


@jax.jit
def kernel(x, edge_index, W_src0, W_dst0, attn0, bias0, res0, W_src1, W_dst1,
           attn1, bias1, gate_W, gate_b, lin_W, lin_b, cls_W, cls_b):
  src = edge_index[0]
  dst = edge_index[1]
  gw = gate_W.reshape(1, -1)
  b0 = bias0.reshape(1, -1)
  b1 = bias1.reshape(1, -1)
  lb = lin_b.reshape(1, -1)
  cb = cls_b.reshape(1, -1)

  fs00, fs01, fd00, fd01, rv0 = _mm(
      x, jnp.concatenate([W_src0, W_dst0, res0], axis=1), True)
  acc0 = _edge_sc(fs00, fs01, fd00, fd01, src, dst, attn0)
  h1, cm1 = _combine(acc0, rv0, b0)
  cs1, num1, den1 = _pool(h1, cm1, gw)

  fs10, fs11, fd10, fd11 = _mm(
      h1, jnp.concatenate([W_src1, W_dst1], axis=1), False)
  acc1 = _edge_sc(fs10, fs11, fd10, fd11, src, dst, attn1)
  h2, cm2 = _combine(acc1, h1, b1)
  cs2, num2, den2 = _pool(h2, cm2, gw)

  return _final(num1, den1, cs1, num2, den2, cs2,
                lin_W, lb, cls_W, cb)
